# baseline (speedup 1.0000x reference)
import functools

import jax
import jax.numpy as jnp
from jax import lax
from jax.experimental import pallas as pl
from jax.experimental.pallas import tpu as pltpu
from jax.experimental.pallas import tpu_sc as plsc

_N = 10000
_E = 320000
_D = 128
_H = _D // 2
_ES = 8192
_NC = 2
_NS = 16
_EPT = _E // _NS
_C = 80
_G = 256
_PADE = _G * _C - _EPT
_NP = _N + 8
_RPT = 624
_ZB = 208


def _edge_scatter(m2, srcr, dstr, with_deg):
    mesh = plsc.VectorSubcoreMesh(core_axis_name="c", subcore_axis_name="s")
    out_type = [jax.ShapeDtypeStruct((_NC, _N, _H), jnp.float32)]
    if with_deg:
        out_type.append(jax.ShapeDtypeStruct((_NC, _N), jnp.float32))
    scratch = [
        pltpu.VMEM((_G, _C), jnp.int32),
        pltpu.VMEM((_G, _C), jnp.int32),
        pltpu.VMEM((_C, _H), jnp.float32),
        pltpu.VMEM((_C, _H), jnp.float32),
        pltpu.VMEM((_C, _H), jnp.float32),
        pltpu.VMEM((_C, _H), jnp.float32),
        pltpu.VMEM((_C, _H), jnp.float32),
        pltpu.VMEM((_C, _H), jnp.float32),
        pltpu.VMEM((_C, _H), jnp.float32),
        pltpu.VMEM((_C, _H), jnp.float32),
        pltpu.VMEM((_ZB, _H), jnp.float32),
    ]
    if with_deg:
        scratch.append(pltpu.VMEM((_N,), jnp.float32))
        scratch.append(pltpu.VMEM((_C,), jnp.float32))
        scratch.append(pltpu.VMEM_SHARED((_NP,), jnp.float32))
    scratch.append(pltpu.VMEM_SHARED((_NP, _H), jnp.float32))
    scratch.append(pltpu.SemaphoreType.DMA)
    scratch.append(pltpu.SemaphoreType.DMA)
    scratch.append(pltpu.SemaphoreType.DMA)

    def body(*refs):
        m_hbm, srcr_hbm, dstr_hbm = refs[:3]
        if with_deg:
            out_hbm, deg_hbm = refs[3:5]
            src_v, dst_v = refs[5:7]
            bufs = refs[7:15]
            (zbuf, degz_v, ones_v, deg_sh, acc, gsem, ssem, dsem) = refs[15:]
        else:
            out_hbm = refs[3]
            src_v, dst_v = refs[4:6]
            bufs = refs[6:14]
            (zbuf, acc, gsem, ssem, dsem) = refs[14:]

        cid = lax.axis_index("c")
        sid = lax.axis_index("s")
        zero16 = jnp.zeros((16,), jnp.float32)
        ones16 = jnp.ones((16,), jnp.float32)

        def _zrow(i, carry):
            for j in range(_H // 16):
                zbuf[i, pl.ds(j * 16, 16)] = zero16
            return carry
        lax.fori_loop(0, _ZB, _zrow, 0)
        for k in range(_RPT // _ZB):
            pltpu.sync_copy(zbuf, acc.at[pl.ds(sid * _RPT + k * _ZB, _ZB)])

        @pl.when(sid == _NS - 1)
        def _zero_tail():
            pltpu.sync_copy(zbuf.at[pl.ds(0, 16)], acc.at[pl.ds(_NS * _RPT, 16)])

        if with_deg:
            def _zdeg(i, carry):
                degz_v[pl.ds(i * 16, 16)] = zero16
                return carry
            lax.fori_loop(0, _N // 16, _zdeg, 0)
            for j in range(_C // 16):
                ones_v[pl.ds(j * 16, 16)] = ones16

            @pl.when(sid == 0)
            def _zero_deg():
                pltpu.sync_copy(degz_v, deg_sh.at[pl.ds(0, _N)])
        plsc.subcore_barrier()

        pltpu.sync_copy(srcr_hbm.at[sid], src_v)
        pltpu.sync_copy(dstr_hbm.at[sid], dst_v)

        my_m = m_hbm.at[cid]

        _K = 2
        A, B = tuple(bufs[:_K]), tuple(bufs[_K:2 * _K])

        def _group(j, a, b, last):
            g = _K * j
            for t in range(_K):
                pltpu.make_async_copy(my_m.at[src_v.at[g + t]], a[t],
                                      gsem).wait()

            @pl.when(j > 0)
            def _drain_prev():
                for t in range(_K):
                    pltpu.make_async_copy(b[t], acc.at[dst_v.at[g]],
                                          ssem).wait()
                if with_deg:
                    for _ in range(_K // 2):
                        pltpu.make_async_copy(
                            ones_v, deg_sh.at[dst_v.at[g]], dsem).wait()

            if not last:
                for t in range(_K):
                    pltpu.async_copy(my_m.at[src_v.at[g + _K + t]], b[t], gsem)
            for t in range(_K):
                pltpu.async_copy(a[t], acc.at[dst_v.at[g + t]], ssem, add=True)
            if with_deg:
                for t in range(_K):
                    @pl.when(lax.rem(g + t, 2) == cid)
                    def _deg_add(gt=g + t):
                        pltpu.async_copy(ones_v, deg_sh.at[dst_v.at[gt]],
                                         dsem, add=True)

        _NG = _G // _K
        for t in range(_K):
            pltpu.async_copy(my_m.at[src_v.at[t]], A[t], gsem)

        def _gpair(i, carry):
            j = i * 2
            _group(j, A, B, False)
            _group(j + 1, B, A, False)
            return carry
        lax.fori_loop(0, _NG // 2 - 1, _gpair, 0)
        _group(_NG - 2, A, B, False)
        _group(_NG - 1, B, A, True)
        for t in range(_K):
            pltpu.make_async_copy(B[t], acc.at[dst_v.at[0]], ssem).wait()
        if with_deg:
            for _ in range(_K // 2):
                pltpu.make_async_copy(ones_v, deg_sh.at[dst_v.at[0]],
                                      dsem).wait()

        plsc.subcore_barrier()
        pltpu.sync_copy(acc.at[pl.ds(sid * _RPT, _RPT)],
                        out_hbm.at[cid].at[pl.ds(sid * _RPT, _RPT)])

        @pl.when(sid == _NS - 1)
        def _write_tail():
            pltpu.sync_copy(acc.at[pl.ds(_NS * _RPT, 16)],
                            out_hbm.at[cid].at[pl.ds(_NS * _RPT, 16)])

        if with_deg:
            @pl.when(sid == 0)
            def _write_deg():
                pltpu.sync_copy(deg_sh.at[pl.ds(0, _N)], degz_v)
                pltpu.sync_copy(degz_v, deg_hbm.at[cid])

    f = pl.kernel(body, out_type=out_type, mesh=mesh, scratch_types=scratch,
                  compiler_params=pltpu.CompilerParams(use_tc_tiling_on_sc=False))
    return f(m2, srcr, dstr)


def _pair_gather(h, ids):
    mesh = plsc.VectorSubcoreMesh(core_axis_name="c", subcore_axis_name="s")
    out_type = [jax.ShapeDtypeStruct((_ES, _D), jnp.float32)] * 2
    scratch = [
        pltpu.VMEM((4, 128), jnp.int32),
        pltpu.VMEM((128, _D), jnp.float32),
        pltpu.SemaphoreType.DMA,
    ]

    def body(h_hbm, ids_hbm, zs_hbm, zd_hbm, idx_v, rows_v, sem):
        cid = lax.axis_index("c")
        sid = lax.axis_index("s")
        wid = sid * _NC + cid
        pltpu.sync_copy(ids_hbm.at[wid], idx_v)
        for a, out_hbm in enumerate((zs_hbm, zd_hbm)):
            for j in range(2):
                pltpu.async_copy(h_hbm.at[idx_v.at[a * 2 + j]], rows_v, sem).wait()
                pltpu.sync_copy(rows_v, out_hbm.at[pl.ds((wid * 2 + j) * 128, 128)])

    f = pl.kernel(body, out_type=out_type, mesh=mesh, scratch_types=scratch)
    return f(h, ids)


def _split_cols(full):
    halves = jnp.stack([full[:, :_H], full[:, _H:]], axis=0)
    return jnp.concatenate(
        [halves, jnp.zeros((_NC, _NP - _N, _H), jnp.float32)], axis=1)


def _mm2(x, wa, wb):
    def body(x_ref, wa_ref, wb_ref, oa_ref, ob_ref):
        xb = x_ref[...]
        oa_ref[...] = _split_cols(
            jnp.dot(xb, wa_ref[...], preferred_element_type=jnp.float32))
        ob_ref[...] = jnp.dot(xb, wb_ref[...], preferred_element_type=jnp.float32)
    return pl.pallas_call(
        body,
        out_shape=[jax.ShapeDtypeStruct((_NC, _NP, _H), jnp.float32),
                   jax.ShapeDtypeStruct((_N, _D), jnp.float32)],
    )(x, wa, wb)


def _gnn_update_mm(xs, s, degp, b, wn, ws):
    def body(xs_ref, s_ref, degp_ref, b_ref, wn_ref, ws_ref, m_ref, hs_ref):
        deg = jnp.maximum(degp_ref[0] + degp_ref[1], 1.0)
        agg = jnp.concatenate([s_ref[0], s_ref[1]], axis=-1)
        h = xs_ref[...] + agg / deg[:, None] + b_ref[...]
        h = jnp.maximum(h, 0.0)
        m_ref[...] = _split_cols(
            jnp.dot(h, wn_ref[...], preferred_element_type=jnp.float32))
        hs_ref[...] = jnp.dot(h, ws_ref[...], preferred_element_type=jnp.float32)
    return pl.pallas_call(
        body,
        out_shape=[jax.ShapeDtypeStruct((_NC, _NP, _H), jnp.float32),
                   jax.ShapeDtypeStruct((_N, _D), jnp.float32)],
    )(xs, s, degp, b, wn, ws)


def _gnn_update_final(hs, s, degp, b):
    def body(hs_ref, s_ref, degp_ref, b_ref, o_ref):
        deg = jnp.maximum(degp_ref[0] + degp_ref[1], 1.0)
        agg = jnp.concatenate([s_ref[0], s_ref[1]], axis=-1)
        h = hs_ref[...] + agg / deg[:, None] + b_ref[...]
        o_ref[...] = jnp.maximum(h, 0.0)
    return pl.pallas_call(
        body,
        out_shape=jax.ShapeDtypeStruct((_N, _D), jnp.float32),
    )(hs, s, degp, b)


def _decoder(zs, zd, w1a, w1b, b1, w2, b2):
    def body(zs_ref, zd_ref, w1a_ref, w1b_ref, b1_ref, w2_ref, b2_ref, o_ref):
        t = jnp.dot(zs_ref[...], w1a_ref[...], preferred_element_type=jnp.float32)
        t = t + jnp.dot(zd_ref[...], w1b_ref[...], preferred_element_type=jnp.float32)
        t = jnp.maximum(t + b1_ref[...], 0.0)
        o_ref[...] = jnp.dot(t, w2_ref[...], preferred_element_type=jnp.float32) + b2_ref[...]
    return pl.pallas_call(
        body,
        out_shape=jax.ShapeDtypeStruct((_ES, _D), jnp.float32),
    )(zs, zd, w1a, w1b, b1, w2, b2)


def kernel(x, edge_index, src_idx, dst_idx, W_self0, W_neigh0, b0,
           W_self1, W_neigh1, b1g, W1, b1, W2, b2):
    pad = jnp.full((_NS, _PADE), _N, jnp.int32)
    srcr = jnp.concatenate(
        [edge_index[0].reshape(_NS, _EPT), pad], axis=1).reshape(_NS, _G, _C)
    dstr = jnp.concatenate(
        [edge_index[1].reshape(_NS, _EPT), pad], axis=1).reshape(_NS, _G, _C)
    ids = jnp.concatenate([src_idx.reshape(_NC * _NS, 2, 128),
                           dst_idx.reshape(_NC * _NS, 2, 128)], axis=1)
    b0r = b0.reshape(1, _D)
    b1gr = b1g.reshape(1, _D)
    b1r = b1.reshape(1, _D)
    b2r = b2.reshape(1, _D)
    w1a = W1[:_D]
    w1b = W1[_D:]

    m0, xs0 = _mm2(x, W_neigh0, W_self0)
    s0, degp = _edge_scatter(m0, srcr, dstr, with_deg=True)
    m1, hs1 = _gnn_update_mm(xs0, s0, degp, b0r, W_neigh1, W_self1)
    s1 = _edge_scatter(m1, srcr, dstr, with_deg=False)[0]
    h2 = _gnn_update_final(hs1, s1, degp, b1gr)
    zs, zd = _pair_gather(h2, ids)
    out = _decoder(zs, zd, w1a, w1b, b1r, W2, b2r)
    return out.reshape(-1)


# speedup vs baseline: 1.1651x; 1.1651x over previous
import functools

import jax
import jax.numpy as jnp
from jax import lax
from jax.experimental import pallas as pl
from jax.experimental.pallas import tpu as pltpu
from jax.experimental.pallas import tpu_sc as plsc

_N = 10000
_E = 320000
_D = 128
_H = _D // 2
_ES = 8192
_NC = 2
_NS = 16
_EPT = _E // _NS
_C = 96
_G = 212
_PADE = _G * _C - _EPT
_NP = _N + 8
_RPT = 624
_ZB = 208


def _edge_scatter(m2, srcr, dstr, with_deg):
    mesh = plsc.VectorSubcoreMesh(core_axis_name="c", subcore_axis_name="s")
    out_type = [jax.ShapeDtypeStruct((_NC, _N, _H), jnp.float32)]
    if with_deg:
        out_type.append(jax.ShapeDtypeStruct((_NC, _N), jnp.float32))
    scratch = [
        pltpu.VMEM((_G, _C), jnp.int32),
        pltpu.VMEM((_G, _C), jnp.int32),
        pltpu.VMEM((_C, _H), jnp.float32),
        pltpu.VMEM((_C, _H), jnp.float32),
        pltpu.VMEM((_C, _H), jnp.float32),
        pltpu.VMEM((_C, _H), jnp.float32),
        pltpu.VMEM((_ZB, _H), jnp.float32),
    ]
    if with_deg:
        scratch.append(pltpu.VMEM((_N,), jnp.float32))
        scratch.append(pltpu.VMEM((_C,), jnp.float32))
        scratch.append(pltpu.VMEM_SHARED((_NP,), jnp.float32))
    scratch.append(pltpu.VMEM_SHARED((_NP, _H), jnp.float32))
    scratch.append(pltpu.SemaphoreType.DMA)
    scratch.append(pltpu.SemaphoreType.DMA)

    def body(*refs):
        m_hbm, srcr_hbm, dstr_hbm = refs[:3]
        if with_deg:
            out_hbm, deg_hbm = refs[3:5]
            src_v, dst_v = refs[5:7]
            bufs = refs[7:11]
            (zbuf, degz_v, ones_v, deg_sh, acc, gsem, ssem) = refs[11:]
        else:
            out_hbm = refs[3]
            src_v, dst_v = refs[4:6]
            bufs = refs[6:10]
            (zbuf, acc, gsem, ssem) = refs[10:]

        cid = lax.axis_index("c")
        sid = lax.axis_index("s")
        zero16 = jnp.zeros((16,), jnp.float32)
        ones16 = jnp.ones((16,), jnp.float32)

        def _zrow(i, carry):
            for j in range(_H // 16):
                zbuf[i, pl.ds(j * 16, 16)] = zero16
            return carry
        lax.fori_loop(0, _ZB, _zrow, 0)
        for k in range(_RPT // _ZB):
            pltpu.sync_copy(zbuf, acc.at[pl.ds(sid * _RPT + k * _ZB, _ZB)])

        @pl.when(sid == _NS - 1)
        def _zero_tail():
            pltpu.sync_copy(zbuf.at[pl.ds(0, 16)], acc.at[pl.ds(_NS * _RPT, 16)])

        if with_deg:
            def _zdeg(i, carry):
                degz_v[pl.ds(i * 16, 16)] = zero16
                return carry
            lax.fori_loop(0, _N // 16, _zdeg, 0)
            for j in range(_C // 16):
                ones_v[pl.ds(j * 16, 16)] = ones16

            @pl.when(sid == 0)
            def _zero_deg():
                pltpu.sync_copy(degz_v, deg_sh.at[pl.ds(0, _N)])
        plsc.subcore_barrier()

        pltpu.sync_copy(srcr_hbm.at[sid], src_v)
        pltpu.sync_copy(dstr_hbm.at[sid], dst_v)

        my_m = m_hbm.at[cid]

        _K = 2
        A, B = tuple(bufs[:_K]), tuple(bufs[_K:2 * _K])

        def _group(j, a, b, last):
            g = _K * j
            for t in range(_K):
                pltpu.make_async_copy(my_m.at[src_v.at[g + t]], a[t],
                                      gsem).wait()

            @pl.when(j > 0)
            def _drain_prev():
                for t in range(_K):
                    pltpu.make_async_copy(b[t], acc.at[dst_v.at[g]],
                                          ssem).wait()

            if not last:
                for t in range(_K):
                    pltpu.async_copy(my_m.at[src_v.at[g + _K + t]], b[t], gsem)
            for t in range(_K):
                pltpu.async_copy(a[t], acc.at[dst_v.at[g + t]], ssem, add=True)
            if with_deg:
                for t in range(_K):
                    @pl.when(lax.rem(g + t, 2) == cid)
                    def _deg_add(gt=g + t):
                        pltpu.sync_copy(ones_v, deg_sh.at[dst_v.at[gt]],
                                        add=True)

        _NG = _G // _K
        for t in range(_K):
            pltpu.async_copy(my_m.at[src_v.at[t]], A[t], gsem)

        def _gpair(i, carry):
            j = i * 2
            _group(j, A, B, False)
            _group(j + 1, B, A, False)
            return carry
        lax.fori_loop(0, _NG // 2 - 1, _gpair, 0)
        _group(_NG - 2, A, B, False)
        _group(_NG - 1, B, A, True)
        for t in range(_K):
            pltpu.make_async_copy(B[t], acc.at[dst_v.at[0]], ssem).wait()

        plsc.subcore_barrier()
        pltpu.sync_copy(acc.at[pl.ds(sid * _RPT, _RPT)],
                        out_hbm.at[cid].at[pl.ds(sid * _RPT, _RPT)])

        @pl.when(sid == _NS - 1)
        def _write_tail():
            pltpu.sync_copy(acc.at[pl.ds(_NS * _RPT, 16)],
                            out_hbm.at[cid].at[pl.ds(_NS * _RPT, 16)])

        if with_deg:
            @pl.when(sid == 0)
            def _write_deg():
                pltpu.sync_copy(deg_sh.at[pl.ds(0, _N)], degz_v)
                pltpu.sync_copy(degz_v, deg_hbm.at[cid])

    f = pl.kernel(body, out_type=out_type, mesh=mesh, scratch_types=scratch,
                  compiler_params=pltpu.CompilerParams(use_tc_tiling_on_sc=False))
    return f(m2, srcr, dstr)


def _pair_gather(h, ids):
    mesh = plsc.VectorSubcoreMesh(core_axis_name="c", subcore_axis_name="s")
    out_type = [jax.ShapeDtypeStruct((_ES, _D), jnp.float32)] * 2
    scratch = [
        pltpu.VMEM((4, 128), jnp.int32),
        pltpu.VMEM((128, _D), jnp.float32),
        pltpu.SemaphoreType.DMA,
    ]

    def body(h_hbm, ids_hbm, zs_hbm, zd_hbm, idx_v, rows_v, sem):
        cid = lax.axis_index("c")
        sid = lax.axis_index("s")
        wid = sid * _NC + cid
        pltpu.sync_copy(ids_hbm.at[wid], idx_v)
        for a, out_hbm in enumerate((zs_hbm, zd_hbm)):
            for j in range(2):
                pltpu.async_copy(h_hbm.at[idx_v.at[a * 2 + j]], rows_v, sem).wait()
                pltpu.sync_copy(rows_v, out_hbm.at[pl.ds((wid * 2 + j) * 128, 128)])

    f = pl.kernel(body, out_type=out_type, mesh=mesh, scratch_types=scratch)
    return f(h, ids)


def _split_cols(full):
    halves = jnp.stack([full[:, :_H], full[:, _H:]], axis=0)
    return jnp.concatenate(
        [halves, jnp.zeros((_NC, _NP - _N, _H), jnp.float32)], axis=1)


def _mm2(x, wa, wb):
    def body(x_ref, wa_ref, wb_ref, oa_ref, ob_ref):
        xb = x_ref[...]
        oa_ref[...] = _split_cols(
            jnp.dot(xb, wa_ref[...], preferred_element_type=jnp.float32))
        ob_ref[...] = jnp.dot(xb, wb_ref[...], preferred_element_type=jnp.float32)
    return pl.pallas_call(
        body,
        out_shape=[jax.ShapeDtypeStruct((_NC, _NP, _H), jnp.float32),
                   jax.ShapeDtypeStruct((_N, _D), jnp.float32)],
    )(x, wa, wb)


def _gnn_update_mm(xs, s, degp, b, wn, ws):
    def body(xs_ref, s_ref, degp_ref, b_ref, wn_ref, ws_ref, m_ref, hs_ref):
        deg = jnp.maximum(degp_ref[0] + degp_ref[1], 1.0)
        agg = jnp.concatenate([s_ref[0], s_ref[1]], axis=-1)
        h = xs_ref[...] + agg / deg[:, None] + b_ref[...]
        h = jnp.maximum(h, 0.0)
        m_ref[...] = _split_cols(
            jnp.dot(h, wn_ref[...], preferred_element_type=jnp.float32))
        hs_ref[...] = jnp.dot(h, ws_ref[...], preferred_element_type=jnp.float32)
    return pl.pallas_call(
        body,
        out_shape=[jax.ShapeDtypeStruct((_NC, _NP, _H), jnp.float32),
                   jax.ShapeDtypeStruct((_N, _D), jnp.float32)],
    )(xs, s, degp, b, wn, ws)


def _gnn_update_final(hs, s, degp, b):
    def body(hs_ref, s_ref, degp_ref, b_ref, o_ref):
        deg = jnp.maximum(degp_ref[0] + degp_ref[1], 1.0)
        agg = jnp.concatenate([s_ref[0], s_ref[1]], axis=-1)
        h = hs_ref[...] + agg / deg[:, None] + b_ref[...]
        o_ref[...] = jnp.maximum(h, 0.0)
    return pl.pallas_call(
        body,
        out_shape=jax.ShapeDtypeStruct((_N, _D), jnp.float32),
    )(hs, s, degp, b)


def _decoder(zs, zd, w1a, w1b, b1, w2, b2):
    def body(zs_ref, zd_ref, w1a_ref, w1b_ref, b1_ref, w2_ref, b2_ref, o_ref):
        t = jnp.dot(zs_ref[...], w1a_ref[...], preferred_element_type=jnp.float32)
        t = t + jnp.dot(zd_ref[...], w1b_ref[...], preferred_element_type=jnp.float32)
        t = jnp.maximum(t + b1_ref[...], 0.0)
        o_ref[...] = jnp.dot(t, w2_ref[...], preferred_element_type=jnp.float32) + b2_ref[...]
    return pl.pallas_call(
        body,
        out_shape=jax.ShapeDtypeStruct((_ES, _D), jnp.float32),
    )(zs, zd, w1a, w1b, b1, w2, b2)


def kernel(x, edge_index, src_idx, dst_idx, W_self0, W_neigh0, b0,
           W_self1, W_neigh1, b1g, W1, b1, W2, b2):
    pad = jnp.full((_NS, _PADE), _N, jnp.int32)
    srcr = jnp.concatenate(
        [edge_index[0].reshape(_NS, _EPT), pad], axis=1).reshape(_NS, _G, _C)
    dstr = jnp.concatenate(
        [edge_index[1].reshape(_NS, _EPT), pad], axis=1).reshape(_NS, _G, _C)
    ids = jnp.concatenate([src_idx.reshape(_NC * _NS, 2, 128),
                           dst_idx.reshape(_NC * _NS, 2, 128)], axis=1)
    b0r = b0.reshape(1, _D)
    b1gr = b1g.reshape(1, _D)
    b1r = b1.reshape(1, _D)
    b2r = b2.reshape(1, _D)
    w1a = W1[:_D]
    w1b = W1[_D:]

    m0, xs0 = _mm2(x, W_neigh0, W_self0)
    s0, degp = _edge_scatter(m0, srcr, dstr, with_deg=True)
    m1, hs1 = _gnn_update_mm(xs0, s0, degp, b0r, W_neigh1, W_self1)
    s1 = _edge_scatter(m1, srcr, dstr, with_deg=False)[0]
    h2 = _gnn_update_final(hs1, s1, degp, b1gr)
    zs, zd = _pair_gather(h2, ids)
    out = _decoder(zs, zd, w1a, w1b, b1r, W2, b2r)
    return out.reshape(-1)


# speedup vs baseline: 1.2119x; 1.0401x over previous
import functools

import jax
import jax.numpy as jnp
from jax import lax
from jax.experimental import pallas as pl
from jax.experimental.pallas import tpu as pltpu
from jax.experimental.pallas import tpu_sc as plsc

_N = 10000
_E = 320000
_D = 128
_H = _D // 2
_ES = 8192
_NC = 2
_NS = 16
_EPT = _E // _NS
_C = 64
_G = 316
_PADE = _G * _C - _EPT
_NP = _N + 8
_RPT = 624
_ZB = 208


def _edge_scatter(m2, srcr, dstr, with_deg):
    mesh = plsc.VectorSubcoreMesh(core_axis_name="c", subcore_axis_name="s")
    out_type = [jax.ShapeDtypeStruct((_NC, _N, _H), jnp.float32)]
    if with_deg:
        out_type.append(jax.ShapeDtypeStruct((_NC, _N), jnp.float32))
    scratch = [
        pltpu.VMEM((_G, _C), jnp.int32),
        pltpu.VMEM((_G, _C), jnp.int32),
        pltpu.VMEM((_C, _H), jnp.float32),
        pltpu.VMEM((_C, _H), jnp.float32),
        pltpu.VMEM((_C, _H), jnp.float32),
        pltpu.VMEM((_C, _H), jnp.float32),
        pltpu.VMEM((_ZB, _H), jnp.float32),
    ]
    if with_deg:
        scratch.append(pltpu.VMEM((_N,), jnp.float32))
        scratch.append(pltpu.VMEM((_C,), jnp.float32))
        scratch.append(pltpu.VMEM_SHARED((_NP,), jnp.float32))
    scratch.append(pltpu.VMEM_SHARED((_NP, _H), jnp.float32))
    scratch.append(pltpu.SemaphoreType.DMA)
    scratch.append(pltpu.SemaphoreType.DMA)

    def body(*refs):
        m_hbm, srcr_hbm, dstr_hbm = refs[:3]
        if with_deg:
            out_hbm, deg_hbm = refs[3:5]
            src_v, dst_v = refs[5:7]
            bufs = refs[7:11]
            (zbuf, degz_v, ones_v, deg_sh, acc, gsem, ssem) = refs[11:]
        else:
            out_hbm = refs[3]
            src_v, dst_v = refs[4:6]
            bufs = refs[6:10]
            (zbuf, acc, gsem, ssem) = refs[10:]

        cid = lax.axis_index("c")
        sid = lax.axis_index("s")
        zero16 = jnp.zeros((16,), jnp.float32)
        ones16 = jnp.ones((16,), jnp.float32)

        def _zrow(i, carry):
            for j in range(_H // 16):
                zbuf[i, pl.ds(j * 16, 16)] = zero16
            return carry
        lax.fori_loop(0, _ZB, _zrow, 0)
        for k in range(_RPT // _ZB):
            pltpu.sync_copy(zbuf, acc.at[pl.ds(sid * _RPT + k * _ZB, _ZB)])

        @pl.when(sid == _NS - 1)
        def _zero_tail():
            pltpu.sync_copy(zbuf.at[pl.ds(0, 16)], acc.at[pl.ds(_NS * _RPT, 16)])

        if with_deg:
            def _zdeg(i, carry):
                degz_v[pl.ds(i * 16, 16)] = zero16
                return carry
            lax.fori_loop(0, _N // 16, _zdeg, 0)
            for j in range(_C // 16):
                ones_v[pl.ds(j * 16, 16)] = ones16

            @pl.when(sid == 0)
            def _zero_deg():
                pltpu.sync_copy(degz_v, deg_sh.at[pl.ds(0, _N)])
        plsc.subcore_barrier()

        pltpu.sync_copy(srcr_hbm.at[sid], src_v)
        pltpu.sync_copy(dstr_hbm.at[sid], dst_v)

        my_m = m_hbm.at[cid]

        _K = 2
        A, B = tuple(bufs[:_K]), tuple(bufs[_K:2 * _K])

        def _group(j, a, b, last):
            g = _K * j
            for t in range(_K):
                pltpu.make_async_copy(my_m.at[src_v.at[g + t]], a[t],
                                      gsem).wait()

            @pl.when(j > 0)
            def _drain_prev():
                for t in range(_K):
                    pltpu.make_async_copy(b[t], acc.at[dst_v.at[g]],
                                          ssem).wait()

            if not last:
                for t in range(_K):
                    pltpu.async_copy(my_m.at[src_v.at[g + _K + t]], b[t], gsem)
            for t in range(_K):
                pltpu.async_copy(a[t], acc.at[dst_v.at[g + t]], ssem, add=True)
            if with_deg:
                for t in range(_K):
                    @pl.when(lax.rem(g + t, 2) == cid)
                    def _deg_add(gt=g + t):
                        pltpu.sync_copy(ones_v, deg_sh.at[dst_v.at[gt]],
                                        add=True)

        _NG = _G // _K
        for t in range(_K):
            pltpu.async_copy(my_m.at[src_v.at[t]], A[t], gsem)

        def _gpair(i, carry):
            j = i * 2
            _group(j, A, B, False)
            _group(j + 1, B, A, False)
            return carry
        lax.fori_loop(0, _NG // 2 - 1, _gpair, 0)
        _group(_NG - 2, A, B, False)
        _group(_NG - 1, B, A, True)
        for t in range(_K):
            pltpu.make_async_copy(B[t], acc.at[dst_v.at[0]], ssem).wait()

        plsc.subcore_barrier()
        pltpu.sync_copy(acc.at[pl.ds(sid * _RPT, _RPT)],
                        out_hbm.at[cid].at[pl.ds(sid * _RPT, _RPT)])

        @pl.when(sid == _NS - 1)
        def _write_tail():
            pltpu.sync_copy(acc.at[pl.ds(_NS * _RPT, 16)],
                            out_hbm.at[cid].at[pl.ds(_NS * _RPT, 16)])

        if with_deg:
            @pl.when(sid == 0)
            def _write_deg():
                pltpu.sync_copy(deg_sh.at[pl.ds(0, _N)], degz_v)
                pltpu.sync_copy(degz_v, deg_hbm.at[cid])

    f = pl.kernel(body, out_type=out_type, mesh=mesh, scratch_types=scratch,
                  compiler_params=pltpu.CompilerParams(use_tc_tiling_on_sc=False))
    return f(m2, srcr, dstr)


def _pair_gather(h, ids):
    mesh = plsc.VectorSubcoreMesh(core_axis_name="c", subcore_axis_name="s")
    out_type = [jax.ShapeDtypeStruct((_ES, _D), jnp.float32)] * 2
    scratch = [
        pltpu.VMEM((4, 128), jnp.int32),
        pltpu.VMEM((128, _D), jnp.float32),
        pltpu.SemaphoreType.DMA,
    ]

    def body(h_hbm, ids_hbm, zs_hbm, zd_hbm, idx_v, rows_v, sem):
        cid = lax.axis_index("c")
        sid = lax.axis_index("s")
        wid = sid * _NC + cid
        pltpu.sync_copy(ids_hbm.at[wid], idx_v)
        for a, out_hbm in enumerate((zs_hbm, zd_hbm)):
            for j in range(2):
                pltpu.async_copy(h_hbm.at[idx_v.at[a * 2 + j]], rows_v, sem).wait()
                pltpu.sync_copy(rows_v, out_hbm.at[pl.ds((wid * 2 + j) * 128, 128)])

    f = pl.kernel(body, out_type=out_type, mesh=mesh, scratch_types=scratch)
    return f(h, ids)


def _split_cols(full):
    halves = jnp.stack([full[:, :_H], full[:, _H:]], axis=0)
    return jnp.concatenate(
        [halves, jnp.zeros((_NC, _NP - _N, _H), jnp.float32)], axis=1)


def _mm2(x, wa, wb):
    def body(x_ref, wa_ref, wb_ref, oa_ref, ob_ref):
        xb = x_ref[...]
        oa_ref[...] = _split_cols(
            jnp.dot(xb, wa_ref[...], preferred_element_type=jnp.float32))
        ob_ref[...] = jnp.dot(xb, wb_ref[...], preferred_element_type=jnp.float32)
    return pl.pallas_call(
        body,
        out_shape=[jax.ShapeDtypeStruct((_NC, _NP, _H), jnp.float32),
                   jax.ShapeDtypeStruct((_N, _D), jnp.float32)],
    )(x, wa, wb)


def _gnn_update_mm(xs, s, degp, b, wn, ws):
    def body(xs_ref, s_ref, degp_ref, b_ref, wn_ref, ws_ref, m_ref, hs_ref):
        deg = jnp.maximum(degp_ref[0] + degp_ref[1], 1.0)
        agg = jnp.concatenate([s_ref[0], s_ref[1]], axis=-1)
        h = xs_ref[...] + agg / deg[:, None] + b_ref[...]
        h = jnp.maximum(h, 0.0)
        m_ref[...] = _split_cols(
            jnp.dot(h, wn_ref[...], preferred_element_type=jnp.float32))
        hs_ref[...] = jnp.dot(h, ws_ref[...], preferred_element_type=jnp.float32)
    return pl.pallas_call(
        body,
        out_shape=[jax.ShapeDtypeStruct((_NC, _NP, _H), jnp.float32),
                   jax.ShapeDtypeStruct((_N, _D), jnp.float32)],
    )(xs, s, degp, b, wn, ws)


def _gnn_update_final(hs, s, degp, b):
    def body(hs_ref, s_ref, degp_ref, b_ref, o_ref):
        deg = jnp.maximum(degp_ref[0] + degp_ref[1], 1.0)
        agg = jnp.concatenate([s_ref[0], s_ref[1]], axis=-1)
        h = hs_ref[...] + agg / deg[:, None] + b_ref[...]
        o_ref[...] = jnp.maximum(h, 0.0)
    return pl.pallas_call(
        body,
        out_shape=jax.ShapeDtypeStruct((_N, _D), jnp.float32),
    )(hs, s, degp, b)


def _decoder(zs, zd, w1a, w1b, b1, w2, b2):
    def body(zs_ref, zd_ref, w1a_ref, w1b_ref, b1_ref, w2_ref, b2_ref, o_ref):
        t = jnp.dot(zs_ref[...], w1a_ref[...], preferred_element_type=jnp.float32)
        t = t + jnp.dot(zd_ref[...], w1b_ref[...], preferred_element_type=jnp.float32)
        t = jnp.maximum(t + b1_ref[...], 0.0)
        o_ref[...] = jnp.dot(t, w2_ref[...], preferred_element_type=jnp.float32) + b2_ref[...]
    return pl.pallas_call(
        body,
        out_shape=jax.ShapeDtypeStruct((_ES, _D), jnp.float32),
    )(zs, zd, w1a, w1b, b1, w2, b2)


def kernel(x, edge_index, src_idx, dst_idx, W_self0, W_neigh0, b0,
           W_self1, W_neigh1, b1g, W1, b1, W2, b2):
    pad = jnp.full((_NS, _PADE), _N, jnp.int32)
    srcr = jnp.concatenate(
        [edge_index[0].reshape(_NS, _EPT), pad], axis=1).reshape(_NS, _G, _C)
    dstr = jnp.concatenate(
        [edge_index[1].reshape(_NS, _EPT), pad], axis=1).reshape(_NS, _G, _C)
    ids = jnp.concatenate([src_idx.reshape(_NC * _NS, 2, 128),
                           dst_idx.reshape(_NC * _NS, 2, 128)], axis=1)
    b0r = b0.reshape(1, _D)
    b1gr = b1g.reshape(1, _D)
    b1r = b1.reshape(1, _D)
    b2r = b2.reshape(1, _D)
    w1a = W1[:_D]
    w1b = W1[_D:]

    m0, xs0 = _mm2(x, W_neigh0, W_self0)
    s0, degp = _edge_scatter(m0, srcr, dstr, with_deg=True)
    m1, hs1 = _gnn_update_mm(xs0, s0, degp, b0r, W_neigh1, W_self1)
    s1 = _edge_scatter(m1, srcr, dstr, with_deg=False)[0]
    h2 = _gnn_update_final(hs1, s1, degp, b1gr)
    zs, zd = _pair_gather(h2, ids)
    out = _decoder(zs, zd, w1a, w1b, b1r, W2, b2r)
    return out.reshape(-1)


# speedup vs baseline: 1.3812x; 1.1397x over previous
import functools

import jax
import jax.numpy as jnp
from jax import lax
from jax.experimental import pallas as pl
from jax.experimental.pallas import tpu as pltpu
from jax.experimental.pallas import tpu_sc as plsc

_N = 10000
_E = 320000
_D = 128
_H = _D // 2
_ES = 8192
_NC = 2
_NS = 16
_EPT = _E // _NS
_C = 80
_G = 252
_PADE = _G * _C - _EPT
_NP = _N + 8
_RPT = 624
_ZB = 208


def _edge_scatter(m2, srcr, dstr, with_deg):
    mesh = plsc.VectorSubcoreMesh(core_axis_name="c", subcore_axis_name="s")
    out_type = [jax.ShapeDtypeStruct((_NC, _N, _H), jnp.float32)]
    if with_deg:
        out_type.append(jax.ShapeDtypeStruct((_NC, _N), jnp.float32))
    scratch = [
        pltpu.VMEM((_G, _C), jnp.int32),
        pltpu.VMEM((_G, _C), jnp.int32),
        pltpu.VMEM((_C, _H), jnp.float32),
        pltpu.VMEM((_C, _H), jnp.float32),
        pltpu.VMEM((_C, _H), jnp.float32),
        pltpu.VMEM((_C, _H), jnp.float32),
        pltpu.VMEM((_ZB, _H), jnp.float32),
    ]
    if with_deg:
        scratch.append(pltpu.VMEM((_N,), jnp.float32))
        scratch.append(pltpu.VMEM((_C,), jnp.float32))
        scratch.append(pltpu.VMEM_SHARED((_NP,), jnp.float32))
    scratch.append(pltpu.VMEM_SHARED((_NP, _H), jnp.float32))
    scratch.append(pltpu.SemaphoreType.DMA)
    scratch.append(pltpu.SemaphoreType.DMA)

    def body(*refs):
        m_hbm, srcr_hbm, dstr_hbm = refs[:3]
        if with_deg:
            out_hbm, deg_hbm = refs[3:5]
            src_v, dst_v = refs[5:7]
            bufs = refs[7:11]
            (zbuf, degz_v, ones_v, deg_sh, acc, gsem, ssem) = refs[11:]
        else:
            out_hbm = refs[3]
            src_v, dst_v = refs[4:6]
            bufs = refs[6:10]
            (zbuf, acc, gsem, ssem) = refs[10:]

        cid = lax.axis_index("c")
        sid = lax.axis_index("s")
        zero16 = jnp.zeros((16,), jnp.float32)
        ones16 = jnp.ones((16,), jnp.float32)

        def _zrow(i, carry):
            for j in range(_H // 16):
                zbuf[i, pl.ds(j * 16, 16)] = zero16
            return carry
        lax.fori_loop(0, _ZB, _zrow, 0)
        for k in range(_RPT // _ZB):
            pltpu.sync_copy(zbuf, acc.at[pl.ds(sid * _RPT + k * _ZB, _ZB)])

        @pl.when(sid == _NS - 1)
        def _zero_tail():
            pltpu.sync_copy(zbuf.at[pl.ds(0, 16)], acc.at[pl.ds(_NS * _RPT, 16)])

        if with_deg:
            def _zdeg(i, carry):
                degz_v[pl.ds(i * 16, 16)] = zero16
                return carry
            lax.fori_loop(0, _N // 16, _zdeg, 0)
            for j in range(_C // 16):
                ones_v[pl.ds(j * 16, 16)] = ones16

            @pl.when(sid == 0)
            def _zero_deg():
                pltpu.sync_copy(degz_v, deg_sh.at[pl.ds(0, _N)])
        plsc.subcore_barrier()

        pltpu.sync_copy(srcr_hbm.at[sid], src_v)
        pltpu.sync_copy(dstr_hbm.at[sid], dst_v)

        my_m = m_hbm.at[cid]

        _K = 2
        A, B = tuple(bufs[:_K]), tuple(bufs[_K:2 * _K])

        def _group(j, a, b, last):
            g = _K * j
            for t in range(_K):
                pltpu.make_async_copy(my_m.at[src_v.at[g + t]], a[t],
                                      gsem).wait()

            @pl.when(j > 0)
            def _drain_prev():
                for t in range(_K):
                    pltpu.make_async_copy(b[t], acc.at[dst_v.at[g]],
                                          ssem).wait()

            if not last:
                for t in range(_K):
                    pltpu.async_copy(my_m.at[src_v.at[g + _K + t]], b[t], gsem)
            for t in range(_K):
                pltpu.async_copy(a[t], acc.at[dst_v.at[g + t]], ssem, add=True)
            if with_deg:
                for t in range(_K):
                    @pl.when(lax.rem(g + t, 2) == cid)
                    def _deg_add(gt=g + t):
                        pltpu.sync_copy(ones_v, deg_sh.at[dst_v.at[gt]],
                                        add=True)

        _NG = _G // _K
        for t in range(_K):
            pltpu.async_copy(my_m.at[src_v.at[t]], A[t], gsem)

        def _gpair(i, carry):
            j = i * 2
            _group(j, A, B, False)
            _group(j + 1, B, A, False)
            return carry
        lax.fori_loop(0, _NG // 2 - 1, _gpair, 0)
        _group(_NG - 2, A, B, False)
        _group(_NG - 1, B, A, True)
        for t in range(_K):
            pltpu.make_async_copy(B[t], acc.at[dst_v.at[0]], ssem).wait()

        plsc.subcore_barrier()
        pltpu.sync_copy(acc.at[pl.ds(sid * _RPT, _RPT)],
                        out_hbm.at[cid].at[pl.ds(sid * _RPT, _RPT)])

        @pl.when(sid == _NS - 1)
        def _write_tail():
            pltpu.sync_copy(acc.at[pl.ds(_NS * _RPT, 16)],
                            out_hbm.at[cid].at[pl.ds(_NS * _RPT, 16)])

        if with_deg:
            @pl.when(sid == 0)
            def _write_deg():
                pltpu.sync_copy(deg_sh.at[pl.ds(0, _N)], degz_v)
                pltpu.sync_copy(degz_v, deg_hbm.at[cid])

    f = pl.kernel(body, out_type=out_type, mesh=mesh, scratch_types=scratch,
                  compiler_params=pltpu.CompilerParams(use_tc_tiling_on_sc=False))
    return f(m2, srcr, dstr)


def _pair_gather(h, ids):
    mesh = plsc.VectorSubcoreMesh(core_axis_name="c", subcore_axis_name="s")
    out_type = [jax.ShapeDtypeStruct((_ES, _D), jnp.float32)] * 2
    scratch = [
        pltpu.VMEM((4, 128), jnp.int32),
        pltpu.VMEM((128, _D), jnp.float32),
        pltpu.SemaphoreType.DMA,
    ]

    def body(h_hbm, ids_hbm, zs_hbm, zd_hbm, idx_v, rows_v, sem):
        cid = lax.axis_index("c")
        sid = lax.axis_index("s")
        wid = sid * _NC + cid
        pltpu.sync_copy(ids_hbm.at[wid], idx_v)
        for a, out_hbm in enumerate((zs_hbm, zd_hbm)):
            for j in range(2):
                pltpu.async_copy(h_hbm.at[idx_v.at[a * 2 + j]], rows_v, sem).wait()
                pltpu.sync_copy(rows_v, out_hbm.at[pl.ds((wid * 2 + j) * 128, 128)])

    f = pl.kernel(body, out_type=out_type, mesh=mesh, scratch_types=scratch)
    return f(h, ids)


def _split_cols(full):
    halves = jnp.stack([full[:, :_H], full[:, _H:]], axis=0)
    return jnp.concatenate(
        [halves, jnp.zeros((_NC, _NP - _N, _H), jnp.float32)], axis=1)


def _mm2(x, wa, wb):
    def body(x_ref, wa_ref, wb_ref, oa_ref, ob_ref):
        xb = x_ref[...]
        oa_ref[...] = _split_cols(
            jnp.dot(xb, wa_ref[...], preferred_element_type=jnp.float32))
        ob_ref[...] = jnp.dot(xb, wb_ref[...], preferred_element_type=jnp.float32)
    return pl.pallas_call(
        body,
        out_shape=[jax.ShapeDtypeStruct((_NC, _NP, _H), jnp.float32),
                   jax.ShapeDtypeStruct((_N, _D), jnp.float32)],
    )(x, wa, wb)


def _gnn_update_mm(xs, s, degp, b, wn, ws):
    def body(xs_ref, s_ref, degp_ref, b_ref, wn_ref, ws_ref, m_ref, hs_ref):
        deg = jnp.maximum(degp_ref[0] + degp_ref[1], 1.0)
        agg = jnp.concatenate([s_ref[0], s_ref[1]], axis=-1)
        h = xs_ref[...] + agg / deg[:, None] + b_ref[...]
        h = jnp.maximum(h, 0.0)
        m_ref[...] = _split_cols(
            jnp.dot(h, wn_ref[...], preferred_element_type=jnp.float32))
        hs_ref[...] = jnp.dot(h, ws_ref[...], preferred_element_type=jnp.float32)
    return pl.pallas_call(
        body,
        out_shape=[jax.ShapeDtypeStruct((_NC, _NP, _H), jnp.float32),
                   jax.ShapeDtypeStruct((_N, _D), jnp.float32)],
    )(xs, s, degp, b, wn, ws)


def _gnn_update_final(hs, s, degp, b):
    def body(hs_ref, s_ref, degp_ref, b_ref, o_ref):
        deg = jnp.maximum(degp_ref[0] + degp_ref[1], 1.0)
        agg = jnp.concatenate([s_ref[0], s_ref[1]], axis=-1)
        h = hs_ref[...] + agg / deg[:, None] + b_ref[...]
        o_ref[...] = jnp.maximum(h, 0.0)
    return pl.pallas_call(
        body,
        out_shape=jax.ShapeDtypeStruct((_N, _D), jnp.float32),
    )(hs, s, degp, b)


def _decoder(zs, zd, w1a, w1b, b1, w2, b2):
    def body(zs_ref, zd_ref, w1a_ref, w1b_ref, b1_ref, w2_ref, b2_ref, o_ref):
        t = jnp.dot(zs_ref[...], w1a_ref[...], preferred_element_type=jnp.float32)
        t = t + jnp.dot(zd_ref[...], w1b_ref[...], preferred_element_type=jnp.float32)
        t = jnp.maximum(t + b1_ref[...], 0.0)
        o_ref[...] = jnp.dot(t, w2_ref[...], preferred_element_type=jnp.float32) + b2_ref[...]
    return pl.pallas_call(
        body,
        out_shape=jax.ShapeDtypeStruct((_ES, _D), jnp.float32),
    )(zs, zd, w1a, w1b, b1, w2, b2)


def kernel(x, edge_index, src_idx, dst_idx, W_self0, W_neigh0, b0,
           W_self1, W_neigh1, b1g, W1, b1, W2, b2):
    pad = jnp.full((_NS, _PADE), _N, jnp.int32)
    srcr = jnp.concatenate(
        [edge_index[0].reshape(_NS, _EPT), pad], axis=1).reshape(_NS, _G, _C)
    dstr = jnp.concatenate(
        [edge_index[1].reshape(_NS, _EPT), pad], axis=1).reshape(_NS, _G, _C)
    ids = jnp.concatenate([src_idx.reshape(_NC * _NS, 2, 128),
                           dst_idx.reshape(_NC * _NS, 2, 128)], axis=1)
    b0r = b0.reshape(1, _D)
    b1gr = b1g.reshape(1, _D)
    b1r = b1.reshape(1, _D)
    b2r = b2.reshape(1, _D)
    w1a = W1[:_D]
    w1b = W1[_D:]

    m0, xs0 = _mm2(x, W_neigh0, W_self0)
    s0, degp = _edge_scatter(m0, srcr, dstr, with_deg=True)
    m1, hs1 = _gnn_update_mm(xs0, s0, degp, b0r, W_neigh1, W_self1)
    s1 = _edge_scatter(m1, srcr, dstr, with_deg=False)[0]
    h2 = _gnn_update_final(hs1, s1, degp, b1gr)
    zs, zd = _pair_gather(h2, ids)
    out = _decoder(zs, zd, w1a, w1b, b1r, W2, b2r)
    return out.reshape(-1)


# speedup vs baseline: 1.3832x; 1.0015x over previous
import jax
import jax.numpy as jnp
from jax import lax
from jax.experimental import pallas as pl
from jax.experimental.pallas import tpu as pltpu
from jax.experimental.pallas import tpu_sc as plsc

_N = 10000
_E = 320000
_D = 128
_H = _D // 2
_ES = 8192
_NC = 2
_NS = 16
_EPT = _E // _NS
_C = 80
_G = 252
_PADE = _G * _C - _EPT
_NP = _N + 8
_RPT = 624
_ZB = 208


def _edge_scatter(m2, srcr, dstr, with_deg):
    mesh = plsc.VectorSubcoreMesh(core_axis_name="c", subcore_axis_name="s")
    out_type = [jax.ShapeDtypeStruct((_NC, _N, _H), jnp.float32)]
    if with_deg:
        out_type.append(jax.ShapeDtypeStruct((_NC, _N), jnp.float32))
    scratch = [
        pltpu.VMEM((_G, _C), jnp.int32),
        pltpu.VMEM((_G, _C), jnp.int32),
        pltpu.VMEM((_C, _H), jnp.float32),
        pltpu.VMEM((_C, _H), jnp.float32),
        pltpu.VMEM((_C, _H), jnp.float32),
        pltpu.VMEM((_C, _H), jnp.float32),
        pltpu.VMEM((_ZB, _H), jnp.float32),
    ]
    if with_deg:
        scratch.append(pltpu.VMEM((_N,), jnp.float32))
        scratch.append(pltpu.VMEM((_C,), jnp.float32))
        scratch.append(pltpu.VMEM_SHARED((_NP,), jnp.float32))
    scratch.append(pltpu.VMEM_SHARED((_NP, _H), jnp.float32))
    scratch.append(pltpu.SemaphoreType.DMA)
    scratch.append(pltpu.SemaphoreType.DMA)

    def body(*refs):
        m_hbm, srcr_hbm, dstr_hbm = refs[:3]
        if with_deg:
            out_hbm, deg_hbm = refs[3:5]
            src_v, dst_v = refs[5:7]
            bufs = refs[7:11]
            (zbuf, degz_v, ones_v, deg_sh, acc, gsem, ssem) = refs[11:]
        else:
            out_hbm = refs[3]
            src_v, dst_v = refs[4:6]
            bufs = refs[6:10]
            (zbuf, acc, gsem, ssem) = refs[10:]

        cid = lax.axis_index("c")
        sid = lax.axis_index("s")
        zero16 = jnp.zeros((16,), jnp.float32)
        ones16 = jnp.ones((16,), jnp.float32)

        def _zrow(i, carry):
            for j in range(_H // 16):
                zbuf[i, pl.ds(j * 16, 16)] = zero16
            return carry
        lax.fori_loop(0, _ZB, _zrow, 0)
        for k in range(_RPT // _ZB):
            pltpu.sync_copy(zbuf, acc.at[pl.ds(sid * _RPT + k * _ZB, _ZB)])

        @pl.when(sid == _NS - 1)
        def _zero_tail():
            pltpu.sync_copy(zbuf.at[pl.ds(0, 16)], acc.at[pl.ds(_NS * _RPT, 16)])

        if with_deg:
            def _zdeg(i, carry):
                degz_v[pl.ds(i * 16, 16)] = zero16
                return carry
            lax.fori_loop(0, _N // 16, _zdeg, 0)
            for j in range(_C // 16):
                ones_v[pl.ds(j * 16, 16)] = ones16

            @pl.when(sid == 0)
            def _zero_deg():
                pltpu.sync_copy(degz_v, deg_sh.at[pl.ds(0, _N)])
        plsc.subcore_barrier()

        pltpu.sync_copy(srcr_hbm.at[sid], src_v)
        pltpu.sync_copy(dstr_hbm.at[sid], dst_v)

        my_m = m_hbm.at[cid]

        _K = 2
        A, B = tuple(bufs[:_K]), tuple(bufs[_K:2 * _K])

        def _group(j, a, b, last):
            g = _K * j
            for t in range(_K):
                pltpu.make_async_copy(my_m.at[src_v.at[g + t]], a[t],
                                      gsem).wait()

            @pl.when(j > 0)
            def _drain_prev():
                for t in range(_K):
                    pltpu.make_async_copy(b[t], acc.at[dst_v.at[g]],
                                          ssem).wait()

            if not last:
                for t in range(_K):
                    pltpu.async_copy(my_m.at[src_v.at[g + _K + t]], b[t], gsem)
            for t in range(_K):
                pltpu.async_copy(a[t], acc.at[dst_v.at[g + t]], ssem, add=True)
            if with_deg:
                for t in range(_K):
                    @pl.when(lax.rem(g + t, 2) == cid)
                    def _deg_add(gt=g + t):
                        pltpu.sync_copy(ones_v, deg_sh.at[dst_v.at[gt]],
                                        add=True)

        _NG = _G // _K
        for t in range(_K):
            pltpu.async_copy(my_m.at[src_v.at[t]], A[t], gsem)

        def _gpair(i, carry):
            j = i * 2
            _group(j, A, B, False)
            _group(j + 1, B, A, False)
            return carry
        lax.fori_loop(0, _NG // 2 - 1, _gpair, 0)
        _group(_NG - 2, A, B, False)
        _group(_NG - 1, B, A, True)
        for t in range(_K):
            pltpu.make_async_copy(B[t], acc.at[dst_v.at[0]], ssem).wait()

        plsc.subcore_barrier()
        pltpu.sync_copy(acc.at[pl.ds(sid * _RPT, _RPT)],
                        out_hbm.at[cid].at[pl.ds(sid * _RPT, _RPT)])

        @pl.when(sid == _NS - 1)
        def _write_tail():
            pltpu.sync_copy(acc.at[pl.ds(_NS * _RPT, 16)],
                            out_hbm.at[cid].at[pl.ds(_NS * _RPT, 16)])

        if with_deg:
            @pl.when(sid == 0)
            def _write_deg():
                pltpu.sync_copy(deg_sh.at[pl.ds(0, _N)], degz_v)
                pltpu.sync_copy(degz_v, deg_hbm.at[cid])

    f = pl.kernel(body, out_type=out_type, mesh=mesh, scratch_types=scratch,
                  compiler_params=pltpu.CompilerParams(use_tc_tiling_on_sc=False))
    return f(m2, srcr, dstr)


def _pair_gather(h, ids):
    mesh = plsc.VectorSubcoreMesh(core_axis_name="c", subcore_axis_name="s")
    out_type = [jax.ShapeDtypeStruct((_ES, _D), jnp.float32)] * 2
    scratch = [
        pltpu.VMEM((4, 128), jnp.int32),
        pltpu.VMEM((128, _D), jnp.float32),
        pltpu.SemaphoreType.DMA,
    ]

    def body(h_hbm, ids_hbm, zs_hbm, zd_hbm, idx_v, rows_v, sem):
        cid = lax.axis_index("c")
        sid = lax.axis_index("s")
        wid = sid * _NC + cid
        pltpu.sync_copy(ids_hbm.at[wid], idx_v)
        for a, out_hbm in enumerate((zs_hbm, zd_hbm)):
            for j in range(2):
                pltpu.async_copy(h_hbm.at[idx_v.at[a * 2 + j]], rows_v, sem).wait()
                pltpu.sync_copy(rows_v, out_hbm.at[pl.ds((wid * 2 + j) * 128, 128)])

    f = pl.kernel(body, out_type=out_type, mesh=mesh, scratch_types=scratch)
    return f(h, ids)


def _split_cols(full):
    halves = jnp.stack([full[:, :_H], full[:, _H:]], axis=0)
    return jnp.concatenate(
        [halves, jnp.zeros((_NC, _NP - _N, _H), jnp.float32)], axis=1)


def _mm2(x, wa, wb):
    def body(x_ref, wa_ref, wb_ref, oa_ref, ob_ref):
        xb = x_ref[...]
        oa_ref[...] = _split_cols(
            jnp.dot(xb, wa_ref[...], preferred_element_type=jnp.float32))
        ob_ref[...] = jnp.dot(xb, wb_ref[...], preferred_element_type=jnp.float32)
    return pl.pallas_call(
        body,
        out_shape=[jax.ShapeDtypeStruct((_NC, _NP, _H), jnp.float32),
                   jax.ShapeDtypeStruct((_N, _D), jnp.float32)],
    )(x, wa, wb)


def _gnn_update_mm(xs, s, degp, b, wn, ws):
    def body(xs_ref, s_ref, degp_ref, b_ref, wn_ref, ws_ref, m_ref, hs_ref):
        deg = jnp.maximum(degp_ref[0] + degp_ref[1], 1.0)
        agg = jnp.concatenate([s_ref[0], s_ref[1]], axis=-1)
        h = xs_ref[...] + agg / deg[:, None] + b_ref[...]
        h = jnp.maximum(h, 0.0)
        m_ref[...] = _split_cols(
            jnp.dot(h, wn_ref[...], preferred_element_type=jnp.float32))
        hs_ref[...] = jnp.dot(h, ws_ref[...], preferred_element_type=jnp.float32)
    return pl.pallas_call(
        body,
        out_shape=[jax.ShapeDtypeStruct((_NC, _NP, _H), jnp.float32),
                   jax.ShapeDtypeStruct((_N, _D), jnp.float32)],
    )(xs, s, degp, b, wn, ws)


def _gnn_update_final(hs, s, degp, b):
    def body(hs_ref, s_ref, degp_ref, b_ref, o_ref):
        deg = jnp.maximum(degp_ref[0] + degp_ref[1], 1.0)
        agg = jnp.concatenate([s_ref[0], s_ref[1]], axis=-1)
        h = hs_ref[...] + agg / deg[:, None] + b_ref[...]
        o_ref[...] = jnp.maximum(h, 0.0)
    return pl.pallas_call(
        body,
        out_shape=jax.ShapeDtypeStruct((_N, _D), jnp.float32),
    )(hs, s, degp, b)


def _decoder(zs, zd, w1a, w1b, b1, w2, b2):
    def body(zs_ref, zd_ref, w1a_ref, w1b_ref, b1_ref, w2_ref, b2_ref, o_ref):
        t = jnp.dot(zs_ref[...], w1a_ref[...], preferred_element_type=jnp.float32)
        t = t + jnp.dot(zd_ref[...], w1b_ref[...], preferred_element_type=jnp.float32)
        t = jnp.maximum(t + b1_ref[...], 0.0)
        o_ref[...] = jnp.dot(t, w2_ref[...], preferred_element_type=jnp.float32) + b2_ref[...]
    return pl.pallas_call(
        body,
        out_shape=jax.ShapeDtypeStruct((_ES, _D), jnp.float32),
    )(zs, zd, w1a, w1b, b1, w2, b2)


def kernel(x, edge_index, src_idx, dst_idx, W_self0, W_neigh0, b0,
           W_self1, W_neigh1, b1g, W1, b1, W2, b2):
    pad = jnp.full((_NS, _PADE), _N, jnp.int32)
    srcr = jnp.concatenate(
        [edge_index[0].reshape(_NS, _EPT), pad], axis=1).reshape(_NS, _G, _C)
    dstr = jnp.concatenate(
        [edge_index[1].reshape(_NS, _EPT), pad], axis=1).reshape(_NS, _G, _C)
    ids = jnp.concatenate([src_idx.reshape(_NC * _NS, 2, 128),
                           dst_idx.reshape(_NC * _NS, 2, 128)], axis=1)
    b0r = b0.reshape(1, _D)
    b1gr = b1g.reshape(1, _D)
    b1r = b1.reshape(1, _D)
    b2r = b2.reshape(1, _D)
    w1a = W1[:_D]
    w1b = W1[_D:]

    m0, xs0 = _mm2(x, W_neigh0, W_self0)
    s0, degp = _edge_scatter(m0, srcr, dstr, with_deg=True)
    m1, hs1 = _gnn_update_mm(xs0, s0, degp, b0r, W_neigh1, W_self1)
    s1 = _edge_scatter(m1, srcr, dstr, with_deg=False)[0]
    h2 = _gnn_update_final(hs1, s1, degp, b1gr)
    zs, zd = _pair_gather(h2, ids)
    out = _decoder(zs, zd, w1a, w1b, b1r, W2, b2r)
    return out.reshape(-1)
